# baseline (speedup 1.0000x reference)
import jax
import jax.numpy as jnp
from jax.experimental import pallas as pl

O_N = 10000
NUM_OBJS_P1 = 180
NIMG = 8
EMB = 128
GDIM = 128
HID = 512
NOISE_DIM = 64

KPAD = 184
CROWS = KPAD + NIMG + NOISE_DIM
BLK = 5000
NB = O_N // BLK

_HI = jax.lax.Precision.HIGHEST


def _split(x):
    hi = x.astype(jnp.bfloat16)
    lo = (x - hi.astype(jnp.float32)).astype(jnp.bfloat16)
    return hi, lo


def _prep_kernel(objs_ref, oti_ref, obj_emb_ref, gconv_W_ref, gconv_b_ref,
                 att_W_ref, W1a_ref, W1b_ref, W1c_ref, b1_ref,
                 CChi_ref, CClo_ref):
    objs_l = objs_ref[...]
    oti_l = oti_ref[...]
    ohT_obj = (jax.lax.broadcasted_iota(jnp.int32, (KPAD, O_N), 0)
               == objs_l).astype(jnp.bfloat16)
    ohT_img = (jax.lax.broadcasted_iota(jnp.int32, (NIMG, O_N), 0)
               == oti_l).astype(jnp.bfloat16)
    histT = jax.lax.dot_general(ohT_obj, ohT_img, (((1,), (1,)), ((), ())),
                                preferred_element_type=jnp.float32)
    table_g = jnp.dot(obj_emb_ref[...], gconv_W_ref[...], precision=_HI,
                      preferred_element_type=jnp.float32) + gconv_b_ref[...]
    table_a = jnp.dot(table_g, att_W_ref[...], precision=_HI,
                      preferred_element_type=jnp.float32)
    counts = jax.lax.dot_general(
        histT, jnp.ones((KPAD, 1), jnp.float32),
        (((0,), (0,)), ((), ())), preferred_element_type=jnp.float32)
    counts = jnp.where(counts > 0.0, counts, 1.0)
    gc = jax.lax.dot_general(
        histT, table_a, (((0,), (0,)), ((), ())), precision=_HI,
        preferred_element_type=jnp.float32) / counts
    tg = jnp.tanh(gc)
    sig = jax.nn.sigmoid(jax.lax.dot_general(
        table_g, tg, (((1,), (1,)), ((), ())), precision=_HI,
        preferred_element_type=jnp.float32))
    w = histT * sig
    rep = jax.lax.dot_general(
        w, table_g, (((0,), (0,)), ((), ())), precision=_HI,
        preferred_element_type=jnp.float32)
    A = jnp.dot(table_g, W1a_ref[...], precision=_HI,
                preferred_element_type=jnp.float32)
    Brep = jnp.dot(rep, W1b_ref[...], precision=_HI,
                   preferred_element_type=jnp.float32) + b1_ref[...]
    CC = jnp.concatenate([A, Brep, W1c_ref[...]], axis=0)
    hi, lo = _split(CC)
    CChi_ref[...] = hi
    CClo_ref[...] = lo


def _main_kernel(objs_ref, oti_ref, noiseT_ref, CChi_ref, CClo_ref, W2_ref,
                 b2_ref, out_ref):
    objs_l = objs_ref[0]
    oti_l = oti_ref[0]
    ohT_obj = (jax.lax.broadcasted_iota(jnp.int32, (KPAD, BLK), 0)
               == objs_l).astype(jnp.bfloat16)
    ohT_img = (jax.lax.broadcasted_iota(jnp.int32, (NIMG, BLK), 0)
               == oti_l).astype(jnp.bfloat16)
    M = jnp.concatenate([ohT_obj, ohT_img, noiseT_ref[0]], axis=0)
    hx = jax.lax.dot_general(M, CChi_ref[...], (((0,), (0,)), ((), ())),
                             preferred_element_type=jnp.float32)
    hy = jax.lax.dot_general(M, CClo_ref[...], (((0,), (0,)), ((), ())),
                             preferred_element_type=jnp.float32)
    h = jax.nn.relu(hx + hy)
    h_hi, h_lo = _split(h)
    W2_hi, W2_lo = _split(W2_ref[...])
    out = (jnp.dot(h_hi, W2_hi, preferred_element_type=jnp.float32)
           + jnp.dot(h_lo, W2_hi, preferred_element_type=jnp.float32)
           + jnp.dot(h_hi, W2_lo, preferred_element_type=jnp.float32))
    out_ref[...] = out + b2_ref[...]


@jax.jit
def _run(objs, noise, obj_to_img, obj_emb, gconv_W, gconv_b, att_W,
         box_W1, box_b1, box_W2, box_b2):
    objs_r = objs.astype(jnp.int32).reshape(1, O_N)
    oti_r = obj_to_img.astype(jnp.int32).reshape(1, O_N)
    obj_emb_p = jnp.pad(obj_emb, ((0, KPAD - NUM_OBJS_P1), (0, 0)))
    noiseT = noise.astype(jnp.bfloat16).reshape(NB, BLK, NOISE_DIM).swapaxes(1, 2)

    def full(shape, idx=None):
        if idx is None:
            idx = tuple(0 for _ in shape)
        return pl.BlockSpec(shape, lambda b, _i=idx: _i)

    CChi, CClo = pl.pallas_call(
        _prep_kernel,
        grid=(1,),
        in_specs=[
            full((1, O_N)), full((1, O_N)),
            full((KPAD, EMB)), full((EMB, GDIM)), full((1, GDIM)),
            full((GDIM, GDIM)),
            full((GDIM, HID)),
            full((GDIM, HID), (1, 0)),
            full((NOISE_DIM, HID), (4, 0)),
            full((1, HID)),
        ],
        out_specs=[full((CROWS, HID)), full((CROWS, HID))],
        out_shape=[jax.ShapeDtypeStruct((CROWS, HID), jnp.bfloat16),
                   jax.ShapeDtypeStruct((CROWS, HID), jnp.bfloat16)],
    )(objs_r, oti_r, obj_emb_p, gconv_W, gconv_b.reshape(1, GDIM), att_W,
      box_W1, box_W1, box_W1, box_b1.reshape(1, HID))

    objs_b = objs.astype(jnp.int32).reshape(NB, 1, BLK)
    oti_b = obj_to_img.astype(jnp.int32).reshape(NB, 1, BLK)
    out = pl.pallas_call(
        _main_kernel,
        grid=(NB,),
        in_specs=[
            pl.BlockSpec((1, 1, BLK), lambda b: (b, 0, 0)),
            pl.BlockSpec((1, 1, BLK), lambda b: (b, 0, 0)),
            pl.BlockSpec((1, NOISE_DIM, BLK), lambda b: (b, 0, 0)),
            full((CROWS, HID)), full((CROWS, HID)),
            full((HID, 4)), full((1, 4)),
        ],
        out_specs=pl.BlockSpec((BLK, 4), lambda b: (b, 0)),
        out_shape=jax.ShapeDtypeStruct((O_N, 4), jnp.float32),
    )(objs_b, oti_b, noiseT, CChi, CClo, box_W2, box_b2.reshape(1, 4))

    return out


def kernel(objs, triples, noise, obj_to_img, obj_emb, pred_emb, gconv_W,
           gconv_b, att_W, box_W1, box_b1, box_W2, box_b2):
    del triples, pred_emb
    return _run(objs, noise, obj_to_img, obj_emb, gconv_W, gconv_b, att_W,
                box_W1, box_b1, box_W2, box_b2)


# speedup vs baseline: 1.0755x; 1.0755x over previous
import jax
import jax.numpy as jnp
from jax.experimental import pallas as pl

O_N = 10000
NUM_OBJS_P1 = 180
NIMG = 8
EMB = 128
GDIM = 128
HID = 512
NOISE_DIM = 64

KPAD = 184
CROWS = KPAD + NIMG + NOISE_DIM
BLK = 5000
NB = O_N // BLK

_HI = jax.lax.Precision.HIGHEST


def _split(x):
    hi = x.astype(jnp.bfloat16)
    lo = (x - hi.astype(jnp.float32)).astype(jnp.bfloat16)
    return hi, lo


def _prep_kernel(objs_ref, oti_ref, obj_emb_ref, gconv_W_ref, gconv_b_ref,
                 att_W_ref, W1a_ref, W1b_ref, W1c_ref, b1_ref,
                 CChi_ref, CClo_ref):
    objs_l = objs_ref[...]
    oti_l = oti_ref[...]
    ohT_obj = (jax.lax.broadcasted_iota(jnp.int32, (KPAD, O_N), 0)
               == objs_l).astype(jnp.bfloat16)
    ohT_img = (jax.lax.broadcasted_iota(jnp.int32, (NIMG, O_N), 0)
               == oti_l).astype(jnp.bfloat16)
    histT = jax.lax.dot_general(ohT_obj, ohT_img, (((1,), (1,)), ((), ())),
                                preferred_element_type=jnp.float32)
    table_g = jnp.dot(obj_emb_ref[...], gconv_W_ref[...], precision=_HI,
                      preferred_element_type=jnp.float32) + gconv_b_ref[...]
    table_a = jnp.dot(table_g, att_W_ref[...], precision=_HI,
                      preferred_element_type=jnp.float32)
    counts = jax.lax.dot_general(
        histT, jnp.ones((KPAD, 1), jnp.float32),
        (((0,), (0,)), ((), ())), preferred_element_type=jnp.float32)
    counts = jnp.where(counts > 0.0, counts, 1.0)
    gc = jax.lax.dot_general(
        histT, table_a, (((0,), (0,)), ((), ())), precision=_HI,
        preferred_element_type=jnp.float32) / counts
    tg = jnp.tanh(gc)
    sig = jax.nn.sigmoid(jax.lax.dot_general(
        table_g, tg, (((1,), (1,)), ((), ())), precision=_HI,
        preferred_element_type=jnp.float32))
    w = histT * sig
    rep = jax.lax.dot_general(
        w, table_g, (((0,), (0,)), ((), ())), precision=_HI,
        preferred_element_type=jnp.float32)
    A = jnp.dot(table_g, W1a_ref[...], precision=_HI,
                preferred_element_type=jnp.float32)
    Brep = jnp.dot(rep, W1b_ref[...], precision=_HI,
                   preferred_element_type=jnp.float32) + b1_ref[...]
    CC = jnp.concatenate([A, Brep, W1c_ref[...]], axis=0)
    hi, lo = _split(CC)
    CChi_ref[...] = hi
    CClo_ref[...] = lo


def _main_kernel(objs_ref, oti_ref, noiseT_ref, CChi_ref, CClo_ref, W2_ref,
                 b2_ref, out_ref):
    objs_l = objs_ref[0]
    oti_l = oti_ref[0]
    ohT_obj = (jax.lax.broadcasted_iota(jnp.int32, (KPAD, BLK), 0)
               == objs_l).astype(jnp.bfloat16)
    ohT_img = (jax.lax.broadcasted_iota(jnp.int32, (NIMG, BLK), 0)
               == oti_l).astype(jnp.bfloat16)
    M = jnp.concatenate([ohT_obj, ohT_img, noiseT_ref[0]], axis=0)
    hx = jax.lax.dot_general(M, CChi_ref[...], (((0,), (0,)), ((), ())),
                             preferred_element_type=jnp.float32)
    hy = jax.lax.dot_general(M, CClo_ref[...], (((0,), (0,)), ((), ())),
                             preferred_element_type=jnp.float32)
    h = jax.nn.relu(hx + hy)
    h_hi = h.astype(jnp.bfloat16)
    W2_hi, W2_lo = _split(W2_ref[...])
    out = (jnp.dot(h_hi, W2_hi, preferred_element_type=jnp.float32)
           + jnp.dot(h_hi, W2_lo, preferred_element_type=jnp.float32))
    out_ref[...] = out + b2_ref[...]


@jax.jit
def _run(objs, noise, obj_to_img, obj_emb, gconv_W, gconv_b, att_W,
         box_W1, box_b1, box_W2, box_b2):
    objs_r = objs.astype(jnp.int32).reshape(1, O_N)
    oti_r = obj_to_img.astype(jnp.int32).reshape(1, O_N)
    obj_emb_p = jnp.pad(obj_emb, ((0, KPAD - NUM_OBJS_P1), (0, 0)))
    noiseT = noise.astype(jnp.bfloat16).reshape(NB, BLK, NOISE_DIM).swapaxes(1, 2)

    def full(shape, idx=None):
        if idx is None:
            idx = tuple(0 for _ in shape)
        return pl.BlockSpec(shape, lambda b, _i=idx: _i)

    CChi, CClo = pl.pallas_call(
        _prep_kernel,
        grid=(1,),
        in_specs=[
            full((1, O_N)), full((1, O_N)),
            full((KPAD, EMB)), full((EMB, GDIM)), full((1, GDIM)),
            full((GDIM, GDIM)),
            full((GDIM, HID)),
            full((GDIM, HID), (1, 0)),
            full((NOISE_DIM, HID), (4, 0)),
            full((1, HID)),
        ],
        out_specs=[full((CROWS, HID)), full((CROWS, HID))],
        out_shape=[jax.ShapeDtypeStruct((CROWS, HID), jnp.bfloat16),
                   jax.ShapeDtypeStruct((CROWS, HID), jnp.bfloat16)],
    )(objs_r, oti_r, obj_emb_p, gconv_W, gconv_b.reshape(1, GDIM), att_W,
      box_W1, box_W1, box_W1, box_b1.reshape(1, HID))

    objs_b = objs.astype(jnp.int32).reshape(NB, 1, BLK)
    oti_b = obj_to_img.astype(jnp.int32).reshape(NB, 1, BLK)
    out = pl.pallas_call(
        _main_kernel,
        grid=(NB,),
        in_specs=[
            pl.BlockSpec((1, 1, BLK), lambda b: (b, 0, 0)),
            pl.BlockSpec((1, 1, BLK), lambda b: (b, 0, 0)),
            pl.BlockSpec((1, NOISE_DIM, BLK), lambda b: (b, 0, 0)),
            full((CROWS, HID)), full((CROWS, HID)),
            full((HID, 4)), full((1, 4)),
        ],
        out_specs=pl.BlockSpec((BLK, 4), lambda b: (b, 0)),
        out_shape=jax.ShapeDtypeStruct((O_N, 4), jnp.float32),
    )(objs_b, oti_b, noiseT, CChi, CClo, box_W2, box_b2.reshape(1, 4))

    return out


def kernel(objs, triples, noise, obj_to_img, obj_emb, pred_emb, gconv_W,
           gconv_b, att_W, box_W1, box_b1, box_W2, box_b2):
    del triples, pred_emb
    return _run(objs, noise, obj_to_img, obj_emb, gconv_W, gconv_b, att_W,
                box_W1, box_b1, box_W2, box_b2)


# speedup vs baseline: 1.2516x; 1.1637x over previous
import jax
import jax.numpy as jnp
from jax.experimental import pallas as pl

O_N = 10000
NUM_OBJS_P1 = 180
NIMG = 8
EMB = 128
GDIM = 128
HID = 512
NOISE_DIM = 64

KPAD = 184
CROWS = KPAD + NIMG + NOISE_DIM
BLK = 5000
NB = O_N // BLK

_HI = jax.lax.Precision.HIGHEST


def _split(x):
    hi = x.astype(jnp.bfloat16)
    lo = (x - hi.astype(jnp.float32)).astype(jnp.bfloat16)
    return hi, lo


def _prep_kernel(objs_ref, oti_ref, obj_emb_ref, gconv_W_ref, gconv_b_ref,
                 att_W_ref, W1a_ref, W1b_ref, W1c_ref, b1_ref, CChi_ref):
    objs_l = objs_ref[...]
    oti_l = oti_ref[...]
    ohT_obj = (jax.lax.broadcasted_iota(jnp.int32, (KPAD, O_N), 0)
               == objs_l).astype(jnp.bfloat16)
    ohT_img = (jax.lax.broadcasted_iota(jnp.int32, (NIMG, O_N), 0)
               == oti_l).astype(jnp.bfloat16)
    histT = jax.lax.dot_general(ohT_obj, ohT_img, (((1,), (1,)), ((), ())),
                                preferred_element_type=jnp.float32)
    table_g = jnp.dot(obj_emb_ref[...], gconv_W_ref[...], precision=_HI,
                      preferred_element_type=jnp.float32) + gconv_b_ref[...]
    table_a = jnp.dot(table_g, att_W_ref[...], precision=_HI,
                      preferred_element_type=jnp.float32)
    counts = jax.lax.dot_general(
        histT, jnp.ones((KPAD, 1), jnp.float32),
        (((0,), (0,)), ((), ())), preferred_element_type=jnp.float32)
    counts = jnp.where(counts > 0.0, counts, 1.0)
    gc = jax.lax.dot_general(
        histT, table_a, (((0,), (0,)), ((), ())), precision=_HI,
        preferred_element_type=jnp.float32) / counts
    tg = jnp.tanh(gc)
    sig = jax.nn.sigmoid(jax.lax.dot_general(
        table_g, tg, (((1,), (1,)), ((), ())), precision=_HI,
        preferred_element_type=jnp.float32))
    w = histT * sig
    rep = jax.lax.dot_general(
        w, table_g, (((0,), (0,)), ((), ())), precision=_HI,
        preferred_element_type=jnp.float32)
    A = jnp.dot(table_g, W1a_ref[...], precision=_HI,
                preferred_element_type=jnp.float32)
    Brep = jnp.dot(rep, W1b_ref[...], precision=_HI,
                   preferred_element_type=jnp.float32) + b1_ref[...]
    CC = jnp.concatenate([A, Brep, W1c_ref[...]], axis=0)
    CChi_ref[...] = CC.astype(jnp.bfloat16)


def _main_kernel(objs_ref, oti_ref, noiseT_ref, CChi_ref, W2cat_ref,
                 b2_ref, out_ref):
    objs_l = objs_ref[0]
    oti_l = oti_ref[0]
    ohT_obj = (jax.lax.broadcasted_iota(jnp.int32, (KPAD, BLK), 0)
               == objs_l).astype(jnp.bfloat16)
    ohT_img = (jax.lax.broadcasted_iota(jnp.int32, (NIMG, BLK), 0)
               == oti_l).astype(jnp.bfloat16)
    M = jnp.concatenate([ohT_obj, ohT_img, noiseT_ref[0]], axis=0)
    h = jax.nn.relu(jax.lax.dot_general(
        M, CChi_ref[...], (((0,), (0,)), ((), ())),
        preferred_element_type=jnp.float32))
    res = jnp.dot(h.astype(jnp.bfloat16), W2cat_ref[...],
                  preferred_element_type=jnp.float32)
    out_ref[...] = res[:, 0:4] + res[:, 4:8] + b2_ref[...]


@jax.jit
def _run(objs, noise, obj_to_img, obj_emb, gconv_W, gconv_b, att_W,
         box_W1, box_b1, box_W2, box_b2):
    objs_r = objs.astype(jnp.int32).reshape(1, O_N)
    oti_r = obj_to_img.astype(jnp.int32).reshape(1, O_N)
    obj_emb_p = jnp.pad(obj_emb, ((0, KPAD - NUM_OBJS_P1), (0, 0)))
    noiseT = noise.astype(jnp.bfloat16).reshape(NB, BLK, NOISE_DIM).swapaxes(1, 2)
    W2_hi = box_W2.astype(jnp.bfloat16)
    W2_lo = (box_W2 - W2_hi.astype(jnp.float32)).astype(jnp.bfloat16)
    W2cat = jnp.concatenate([W2_hi, W2_lo], axis=1)

    def full(shape, idx=None):
        if idx is None:
            idx = tuple(0 for _ in shape)
        return pl.BlockSpec(shape, lambda b, _i=idx: _i)

    CChi = pl.pallas_call(
        _prep_kernel,
        grid=(1,),
        in_specs=[
            full((1, O_N)), full((1, O_N)),
            full((KPAD, EMB)), full((EMB, GDIM)), full((1, GDIM)),
            full((GDIM, GDIM)),
            full((GDIM, HID)),
            full((GDIM, HID), (1, 0)),
            full((NOISE_DIM, HID), (4, 0)),
            full((1, HID)),
        ],
        out_specs=full((CROWS, HID)),
        out_shape=jax.ShapeDtypeStruct((CROWS, HID), jnp.bfloat16),
    )(objs_r, oti_r, obj_emb_p, gconv_W, gconv_b.reshape(1, GDIM), att_W,
      box_W1, box_W1, box_W1, box_b1.reshape(1, HID))

    objs_b = objs.astype(jnp.int32).reshape(NB, 1, BLK)
    oti_b = obj_to_img.astype(jnp.int32).reshape(NB, 1, BLK)
    out = pl.pallas_call(
        _main_kernel,
        grid=(NB,),
        in_specs=[
            pl.BlockSpec((1, 1, BLK), lambda b: (b, 0, 0)),
            pl.BlockSpec((1, 1, BLK), lambda b: (b, 0, 0)),
            pl.BlockSpec((1, NOISE_DIM, BLK), lambda b: (b, 0, 0)),
            full((CROWS, HID)),
            full((HID, 8)), full((1, 4)),
        ],
        out_specs=pl.BlockSpec((BLK, 4), lambda b: (b, 0)),
        out_shape=jax.ShapeDtypeStruct((O_N, 4), jnp.float32),
    )(objs_b, oti_b, noiseT, CChi, W2cat, box_b2.reshape(1, 4))

    return out


def kernel(objs, triples, noise, obj_to_img, obj_emb, pred_emb, gconv_W,
           gconv_b, att_W, box_W1, box_b1, box_W2, box_b2):
    del triples, pred_emb
    return _run(objs, noise, obj_to_img, obj_emb, gconv_W, gconv_b, att_W,
                box_W1, box_b1, box_W2, box_b2)
